# Initial kernel scaffold; baseline (speedup 1.0000x reference)
import jax
import jax.numpy as jnp
from jax.experimental import pallas as pl


def kernel(x, token_table, pos_table):
    raise NotImplementedError("write your pallas kernel here")


# speedup vs baseline: 1.2393x; 1.2393x over previous
import functools

import jax
import jax.numpy as jnp
from jax import lax
from jax.experimental import pallas as pl
from jax.experimental.pallas import tpu as pltpu
from jax.experimental.pallas import tpu_sc as plsc

_B = 4096
_S = 200
_D = 32
_NW = 32
_PER_W = (_B * _S) // _NW
_CHUNK = 1600
_NITER = _PER_W // _CHUNK
_GROUP = 100
_NGROUP = _CHUNK // _GROUP


def _emb_body(x_hbm, tok_hbm, pos_hbm, out_hbm, idx_v, rows_v, pos_v, sem):
    cid = lax.axis_index("c")
    sid = lax.axis_index("s")
    wid = sid * 2 + cid

    pltpu.sync_copy(pos_hbm, pos_v)

    @pl.loop(0, _NITER)
    def _it(it):
        pltpu.sync_copy(x_hbm.at[wid, it], idx_v)
        descs = [
            pltpu.async_copy(
                tok_hbm.at[idx_v.at[g]],
                rows_v.at[pl.ds(g * _GROUP, _GROUP)],
                sem,
            )
            for g in range(_NGROUP)
        ]
        for d in descs:
            d.wait()

        @pl.loop(0, _CHUNK)
        def _add(r):
            pr = lax.rem(r, _S)
            for h in range(2):
                plsc.addupdate(
                    rows_v.at[r, pl.ds(h * 16, 16)],
                    pos_v[pr, pl.ds(h * 16, 16)],
                )

        pltpu.sync_copy(
            rows_v, out_hbm.at[pl.ds(wid * _PER_W + it * _CHUNK, _CHUNK)]
        )


@jax.jit
def _emb(x_r, token_table, pos_table):
    mesh = plsc.VectorSubcoreMesh(
        core_axis_name="c", subcore_axis_name="s", num_cores=2, num_subcores=16
    )
    f = pl.kernel(
        _emb_body,
        out_type=jax.ShapeDtypeStruct((_B * _S, _D), jnp.float32),
        mesh=mesh,
        scratch_types=[
            pltpu.VMEM((_NGROUP, _GROUP), jnp.int32),
            pltpu.VMEM((_CHUNK, _D), jnp.float32),
            pltpu.VMEM((_S, _D), jnp.float32),
            pltpu.SemaphoreType.DMA,
        ],
        compiler_params=pltpu.CompilerParams(use_tc_tiling_on_sc=False),
    )
    return f(x_r, token_table, pos_table)


def kernel(x, token_table, pos_table):
    x_r = x.reshape(_NW, _NITER, _NGROUP, _GROUP)
    out = _emb(x_r, token_table, pos_table)
    return out.reshape(_B, _S, _D)


# speedup vs baseline: 1.4864x; 1.1994x over previous
import functools

import jax
import jax.numpy as jnp
from jax import lax
from jax.experimental import pallas as pl
from jax.experimental.pallas import tpu as pltpu
from jax.experimental.pallas import tpu_sc as plsc

_B = 4096
_S = 200
_D = 32
_NW = 32
_PER_W = (_B * _S) // _NW
_CHUNK = 1600
_NITER = _PER_W // _CHUNK
_GROUP = 100
_NGROUP = _CHUNK // _GROUP
_REPS = _CHUNK // _S


def _emb_body(
    x_hbm, tok_hbm, pos_hbm, out_hbm,
    idx_v, rows_v, pos_v,
    gsem0, gsem1, osem0, osem1, isem0, isem1,
):
    cid = lax.axis_index("c")
    sid = lax.axis_index("s")
    wid = sid * 2 + cid
    gsems = (gsem0, gsem1)
    osems = (osem0, osem1)
    isems = (isem0, isem1)

    pltpu.sync_copy(pos_hbm, pos_v)

    def fire_idx(it):
        s = it % 2
        return pltpu.async_copy(x_hbm.at[wid, it], idx_v.at[s], isems[s])

    def fire_gathers(it):
        s = it % 2
        return [
            pltpu.async_copy(
                tok_hbm.at[idx_v.at[s, g]],
                rows_v.at[s, pl.ds(g * _GROUP, _GROUP)],
                gsems[s],
            )
            for g in range(_NGROUP)
        ]

    def add_pos(it):
        s = it % 2

        @pl.loop(0, _S)
        def _row(r):
            for h in range(2):
                pv = pos_v[r, pl.ds(h * 16, 16)]
                for rep in range(_REPS):
                    plsc.addupdate(
                        rows_v.at[s, rep * _S + r, pl.ds(h * 16, 16)], pv
                    )

    def fire_out(it):
        s = it % 2
        return pltpu.async_copy(
            rows_v.at[s],
            out_hbm.at[pl.ds(wid * _PER_W + it * _CHUNK, _CHUNK)],
            osems[s],
        )

    d_idx0 = fire_idx(0)
    d_idx1 = fire_idx(1)
    d_idx0.wait()
    gd = {0: fire_gathers(0)}
    idxd = {1: d_idx1}
    outd = {}

    for it in range(_NITER):
        if it + 1 < _NITER:
            idxd[it + 1].wait()
            if it - 1 >= 0:
                outd[it - 1].wait()
            gd[it + 1] = fire_gathers(it + 1)
        for d in gd.pop(it):
            d.wait()
        if it + 2 < _NITER:
            idxd[it + 2] = fire_idx(it + 2)
        add_pos(it)
        outd[it] = fire_out(it)

    outd[_NITER - 2].wait()
    outd[_NITER - 1].wait()


@jax.jit
def _emb(x_r, token_table, pos_table):
    mesh = plsc.VectorSubcoreMesh(
        core_axis_name="c", subcore_axis_name="s", num_cores=2, num_subcores=16
    )
    f = pl.kernel(
        _emb_body,
        out_type=jax.ShapeDtypeStruct((_B * _S, _D), jnp.float32),
        mesh=mesh,
        scratch_types=[
            pltpu.VMEM((2, _NGROUP, _GROUP), jnp.int32),
            pltpu.VMEM((2, _CHUNK, _D), jnp.float32),
            pltpu.VMEM((_S, _D), jnp.float32),
            pltpu.SemaphoreType.DMA,
            pltpu.SemaphoreType.DMA,
            pltpu.SemaphoreType.DMA,
            pltpu.SemaphoreType.DMA,
            pltpu.SemaphoreType.DMA,
            pltpu.SemaphoreType.DMA,
        ],
        compiler_params=pltpu.CompilerParams(use_tc_tiling_on_sc=False),
    )
    return f(x_r, token_table, pos_table)


def kernel(x, token_table, pos_table):
    x_r = x.reshape(_NW, _NITER, _NGROUP, _GROUP)
    out = _emb(x_r, token_table, pos_table)
    return out.reshape(_B, _S, _D)
